# baseline (speedup 1.0000x reference)
import jax
import jax.numpy as jnp
from jax import lax
from jax.experimental import pallas as pl
from jax.experimental.pallas import tpu as pltpu
from jax.experimental.pallas import tpu_sc as plsc

NC = 2
NS = 16
NW = NC * NS
CH = 128


def _sc_body(table, rt, hids, tids, rels, out,
             rt_v, hidx, tidx, rel_v, h0, h1, t0, t1, out_v, tmp, sem0, sem1):
    nch = hids.shape[1]
    c = lax.axis_index("c")
    s = lax.axis_index("s")
    wid = s * NC + c

    pltpu.sync_copy(rt, rt_v)
    pltpu.sync_copy(hids.at[wid], hidx)
    pltpu.sync_copy(tids.at[wid], tidx)
    pltpu.sync_copy(rels.at[wid], rel_v)

    r0 = [rt_v[0, pl.ds(16 * q, 16)] for q in range(4)]
    rd = [rt_v[1, pl.ds(16 * q, 16)] - r0[q] for q in range(4)]

    hb = (h0, h1)
    tb = (t0, t1)
    sems = (sem0, sem1)

    def start(j, slot):
        ch = pltpu.async_copy(table.at[hidx.at[j]], hb[slot], sems[slot])
        ct = pltpu.async_copy(table.at[tidx.at[j]], tb[slot], sems[slot])
        return ch, ct

    iota16 = lax.iota(jnp.int32, 16)
    pend = start(0, 0)
    for j in range(nch):
        slot = j & 1
        cur = pend
        if j + 1 < nch:
            pend = start(j + 1, slot ^ 1)
        cur[0].wait()
        cur[1].wait()
        base = j * CH

        def group(g, _, slot=slot, base=base):
            gb = g * 16
            rv = rel_v[pl.ds(base + gb, 16)]
            for u in range(16):
                i = gb + u
                relf = rv[u]
                acc = None
                for q in range(4):
                    h = hb[slot][i, pl.ds(16 * q, 16)]
                    t = tb[slot][i, pl.ds(16 * q, 16)]
                    e = (h - t) + (r0[q] + relf * rd[q])
                    acc = e * e if acc is None else acc + e * e
                plsc.store_scatter(
                    tmp, [iota16, jnp.full((16,), u, jnp.int32)], acc)
            tot = None
            for l in range(16):
                row = tmp[l]
                tot = row if tot is None else tot + row
            out_v[pl.ds(base + gb, 16)] = -tot
            return 0

        lax.fori_loop(0, CH // 16, group, 0)

    pltpu.sync_copy(out_v, out.at[wid])


def kernel(itemEmbedding, r_table, head_ids, tail_ids, relation_ids):
    b, k = head_ids.shape
    tot = b * k
    epw = tot // NW
    nch = epw // CH
    f = itemEmbedding.shape[1]

    h = head_ids.astype(jnp.int32).reshape(NW, nch, CH)
    t = tail_ids.astype(jnp.int32).reshape(NW, nch, CH)
    r = relation_ids.astype(jnp.float32).reshape(NW, epw)

    mesh = plsc.VectorSubcoreMesh(core_axis_name="c", subcore_axis_name="s")
    run = pl.kernel(
        _sc_body,
        out_type=jax.ShapeDtypeStruct((NW, epw), jnp.float32),
        mesh=mesh,
        scratch_types=[
            pltpu.VMEM((2, f), jnp.float32),
            pltpu.VMEM((nch, CH), jnp.int32),
            pltpu.VMEM((nch, CH), jnp.int32),
            pltpu.VMEM((epw,), jnp.float32),
            pltpu.VMEM((CH, f), jnp.float32),
            pltpu.VMEM((CH, f), jnp.float32),
            pltpu.VMEM((CH, f), jnp.float32),
            pltpu.VMEM((CH, f), jnp.float32),
            pltpu.VMEM((epw,), jnp.float32),
            pltpu.VMEM((16, 16), jnp.float32),
            pltpu.SemaphoreType.DMA,
            pltpu.SemaphoreType.DMA,
        ],
        compiler_params=pltpu.CompilerParams(
            needs_layout_passes=False, use_tc_tiling_on_sc=False),
    )
    out = run(itemEmbedding, r_table, h, t, r)
    return out.reshape(b, k)
